# baseline (speedup 1.0000x reference)
import jax
import jax.numpy as jnp
from jax.experimental import pallas as pl
from jax.experimental.pallas import tpu as pltpu

B, N, D, H = 8, 1024, 32, 128
NUM_MP_STEPS = 2
INNER_ROUNDS = 5
C = 2
_NW = 8 * NUM_MP_STEPS + 8 + 3


def _row_iota():
    return jax.lax.broadcasted_iota(jnp.int32, (N, 1), 0).astype(jnp.float32)


def _ep_one(a_c, z_c, vval, ws, wt, bscal):
    rowmask = (_row_iota() < vval).astype(jnp.float32)
    colmask = (jax.lax.broadcasted_iota(jnp.int32, (1, N), 1
                                        ).astype(jnp.float32)
               < vval).astype(jnp.float32)
    zs = z_c @ ws
    zt = z_c @ wt
    logits = jax.lax.dot_general(
        zs, zt, (((1,), (1,)), ((), ())),
        preferred_element_type=jnp.float32) + bscal
    sp = jnp.maximum(logits, 0.0) + jnp.log1p(jnp.exp(-jnp.abs(logits)))
    logp = a_c * logits - sp
    msum = jnp.sum(logp * rowmask * colmask)
    cnt = jnp.sum(rowmask)
    return msum / (cnt * cnt)


def _body(x_ref, a_ref, aprev_ref, eps_ref, v_ref, vprev_ref, *refs):
    w = [r[...] for r in refs[:_NW]]
    z_ref, kl_ref, ep_ref, zscr = refs[_NW:]
    Ws, Wt, bb = w[_NW - 3:]

    b = pl.program_id(0)

    @pl.when(b == 0)
    def _init():
        zscr[...] = jnp.zeros_like(zscr)

    zprev = zscr[...]
    for c in range(C):
        epv = _ep_one(aprev_ref[c], zprev[c * N:(c + 1) * N],
                      vprev_ref[c, 0, 0], Ws, Wt, bb[0, 0])
        ep_ref[pl.ds(c, 1)] = jnp.full((1, 1, 128), epv, jnp.float32)

    xb = x_ref[...].reshape(C * N, D)
    i = 0
    for _ in range(NUM_MP_STEPS):
        Wm1, bm1, Wm2, bm2, Wu1, bu1, Wu2, bu2 = w[i:i + 8]
        i += 8
        for _ in range(INNER_ROUNDS):
            m = jnp.tanh(jnp.tanh(xb @ Wm1 + bm1) @ Wm2 + bm2)
            aggs = [
                jax.lax.dot(a_ref[c], m[c * N:(c + 1) * N],
                            preferred_element_type=jnp.float32)
                for c in range(C)
            ]
            agg = jnp.concatenate(aggs, axis=0)
            u = jnp.concatenate([xb, agg], axis=1)
            pre = u @ Wu1 + bu1
            xb = xb + jnp.tanh(jnp.tanh(pre) @ Wu2 + bu2)

    W1, b1, W2, b2, W3m, b3m, W3s, b3s = w[i:i + 8]

    h = jnp.tanh(xb @ W1 + b1)
    h = jnp.tanh(h @ W2 + b2)
    mean = h @ W3m + b3m
    log_sd = h @ W3s + b3s
    sd = jnp.exp(log_sd)
    kl = -log_sd + 0.5 * (sd * sd + mean * mean) - 0.5

    z = mean + sd * eps_ref[...].reshape(C * N, D)
    z_ref[...] = z.reshape(C, N, D)
    zscr[...] = z

    iota_col = _row_iota()
    for c in range(C):
        vval = v_ref[c, 0, 0]
        rowmask = (iota_col < vval).astype(jnp.float32)
        klsum = jnp.sum(kl[c * N:(c + 1) * N] * rowmask)
        neg_kl = -(klsum / (N * D)) * vval
        kl_ref[pl.ds(c, 1)] = jnp.full((1, 1, 128), neg_kl, jnp.float32)


def _ep_tail_body(a_ref, z_ref, v_ref, ws_ref, wt_ref, bb_ref, ep_ref):
    for c in range(C):
        epv = _ep_one(a_ref[c], z_ref[c], v_ref[c, 0, 0],
                      ws_ref[...], wt_ref[...], bb_ref[0, 0])
        ep_ref[pl.ds(c, 1)] = jnp.full((1, 1, 128), epv, jnp.float32)


def _full_spec(shape):
    nd = len(shape)
    return pl.BlockSpec(shape, lambda b, _nd=nd: (0,) * _nd)


def kernel(x, a, v, params, eps):
    weights = []
    for p in params['mp']:
        weights += [
            p['Wm1'], p['bm1'].reshape(1, H), p['Wm2'], p['bm2'].reshape(1, D),
            p['Wu1'], p['bu1'].reshape(1, H),
            p['Wu2'], p['bu2'].reshape(1, D),
        ]
    e = params['enc']
    weights += [
        e['W1'], e['b1'].reshape(1, H), e['W2'], e['b2'].reshape(1, H),
        e['W3'][:, :D], e['b3'][:D].reshape(1, D),
        e['W3'][:, D:], e['b3'][D:].reshape(1, D),
    ]
    ep = params['ep']
    ws_w, wt_w = ep['Ws'], ep['Wt']
    bb_w = jnp.broadcast_to(ep['b'].reshape(1, 1), (1, 128))
    weights += [ws_w, wt_w, bb_w]

    vb = jnp.broadcast_to(v.reshape(B, 1, 1), (B, 1, 128))
    nchunk = B // C

    in_specs = [
        pl.BlockSpec((C, N, D), lambda b: (b, 0, 0)),
        pl.BlockSpec((C, N, N), lambda b: (b, 0, 0)),
        pl.BlockSpec((C, N, N), lambda b: (jnp.maximum(b - 1, 0), 0, 0)),
        pl.BlockSpec((C, N, D), lambda b: (b, 0, 0)),
        pl.BlockSpec((C, 1, 128), lambda b: (b, 0, 0)),
        pl.BlockSpec((C, 1, 128), lambda b: (jnp.maximum(b - 1, 0), 0, 0)),
    ] + [_full_spec(wi.shape) for wi in weights]

    out_specs = [
        pl.BlockSpec((C, N, D), lambda b: (b, 0, 0)),
        pl.BlockSpec((C, 1, 128), lambda b: (b, 0, 0)),
        pl.BlockSpec((C, 1, 128), lambda b: (jnp.maximum(b - 1, 0), 0, 0)),
    ]
    out_shape = [
        jax.ShapeDtypeStruct((B, N, D), jnp.float32),
        jax.ShapeDtypeStruct((B, 1, 128), jnp.float32),
        jax.ShapeDtypeStruct((B, 1, 128), jnp.float32),
    ]

    z, klp, epp = pl.pallas_call(
        _body,
        grid=(nchunk,),
        in_specs=in_specs,
        out_specs=out_specs,
        out_shape=out_shape,
        scratch_shapes=[pltpu.VMEM((C * N, D), jnp.float32)],
        compiler_params=pltpu.CompilerParams(
            dimension_semantics=("arbitrary",),
            vmem_limit_bytes=110 * 1024 * 1024,
        ),
    )(x, a, a, eps, vb, vb, *weights)

    ep_tail = pl.pallas_call(
        _ep_tail_body,
        grid=(1,),
        in_specs=[
            pl.BlockSpec((C, N, N), lambda b: (nchunk - 1, 0, 0)),
            pl.BlockSpec((C, N, D), lambda b: (nchunk - 1, 0, 0)),
            pl.BlockSpec((C, 1, 128), lambda b: (nchunk - 1, 0, 0)),
            _full_spec(ws_w.shape),
            _full_spec(wt_w.shape),
            _full_spec(bb_w.shape),
        ],
        out_specs=pl.BlockSpec((C, 1, 128), lambda b: (0, 0, 0)),
        out_shape=jax.ShapeDtypeStruct((C, 1, 128), jnp.float32),
        compiler_params=pltpu.CompilerParams(
            dimension_semantics=("arbitrary",),
            vmem_limit_bytes=110 * 1024 * 1024,
        ),
    )(a, z, vb, ws_w, wt_w, bb_w)

    ep_out = jnp.concatenate([epp[:B - C, 0, 0], ep_tail[:, 0, 0]])
    return (z, klp[:, 0, 0], ep_out)


# speedup vs baseline: 1.0811x; 1.0811x over previous
import jax
import jax.numpy as jnp
from jax.experimental import pallas as pl
from jax.experimental.pallas import tpu as pltpu

B, N, D, H = 8, 1024, 32, 128
NUM_MP_STEPS = 2
INNER_ROUNDS = 5
C = 2
_NW = 8 * NUM_MP_STEPS + 8 + 3


def _body(x_ref, a_ref, eps_ref, v_ref, *refs):
    w = [r[...] for r in refs[:_NW]]
    z_ref, kl_ref, ep_ref = refs[_NW:]

    xb = x_ref[...].reshape(C * N, D)
    i = 0
    for _ in range(NUM_MP_STEPS):
        Wm1, bm1, Wm2, bm2, Wu1, bu1, Wu2, bu2 = w[i:i + 8]
        i += 8
        for _ in range(INNER_ROUNDS):
            m = jnp.tanh(jnp.tanh(xb @ Wm1 + bm1) @ Wm2 + bm2)
            aggs = [
                jax.lax.dot(a_ref[c], m[c * N:(c + 1) * N],
                            preferred_element_type=jnp.float32)
                for c in range(C)
            ]
            agg = jnp.concatenate(aggs, axis=0)
            u = jnp.concatenate([xb, agg], axis=1)
            pre = u @ Wu1 + bu1
            xb = xb + jnp.tanh(jnp.tanh(pre) @ Wu2 + bu2)

    W1, b1, W2, b2, W3m, b3m, W3s, b3s = w[i:i + 8]
    Ws, Wt, bb = w[i + 8:i + 11]

    h = jnp.tanh(xb @ W1 + b1)
    h = jnp.tanh(h @ W2 + b2)
    mean = h @ W3m + b3m
    log_sd = h @ W3s + b3s
    sd = jnp.exp(log_sd)
    kl = -log_sd + 0.5 * (sd * sd + mean * mean) - 0.5

    z = mean + sd * eps_ref[...].reshape(C * N, D)
    z_ref[...] = z.reshape(C, N, D)

    iota_col = jax.lax.broadcasted_iota(jnp.int32, (N, 1), 0
                                        ).astype(jnp.float32)
    iota_row = jax.lax.broadcasted_iota(jnp.int32, (1, N), 1
                                        ).astype(jnp.float32)
    for c in range(C):
        vval = v_ref[c, 0, 0]
        rowmask = (iota_col < vval).astype(jnp.float32)
        klsum = jnp.sum(kl[c * N:(c + 1) * N] * rowmask)
        neg_kl = -(klsum / (N * D)) * vval
        kl_ref[pl.ds(c, 1)] = jnp.full((1, 1, 128), neg_kl, jnp.float32)

        zc = z[c * N:(c + 1) * N]
        zs = zc @ Ws
        zt = zc @ Wt
        logits = jax.lax.dot_general(
            zs, zt, (((1,), (1,)), ((), ())),
            preferred_element_type=jnp.float32) + bb[0, 0]
        sp = jnp.maximum(logits, 0.0) + jnp.log1p(jnp.exp(-jnp.abs(logits)))
        logp = a_ref[c] * logits - sp
        colmask = (iota_row < vval).astype(jnp.float32)
        msum = jnp.sum(logp * rowmask * colmask)
        cnt = jnp.sum(rowmask)
        ep_ref[pl.ds(c, 1)] = jnp.full((1, 1, 128), msum / (cnt * cnt),
                                       jnp.float32)


def _full_spec(shape):
    nd = len(shape)
    return pl.BlockSpec(shape, lambda b, _nd=nd: (0,) * _nd)


def kernel(x, a, v, params, eps):
    weights = []
    for p in params['mp']:
        weights += [
            p['Wm1'], p['bm1'].reshape(1, H), p['Wm2'], p['bm2'].reshape(1, D),
            p['Wu1'], p['bu1'].reshape(1, H),
            p['Wu2'], p['bu2'].reshape(1, D),
        ]
    e = params['enc']
    weights += [
        e['W1'], e['b1'].reshape(1, H), e['W2'], e['b2'].reshape(1, H),
        e['W3'][:, :D], e['b3'][:D].reshape(1, D),
        e['W3'][:, D:], e['b3'][D:].reshape(1, D),
    ]
    ep = params['ep']
    weights += [ep['Ws'], ep['Wt'],
                jnp.broadcast_to(ep['b'].reshape(1, 1), (1, 128))]

    vb = jnp.broadcast_to(v.reshape(B, 1, 1), (B, 1, 128))

    in_specs = [
        pl.BlockSpec((C, N, D), lambda b: (b, 0, 0)),
        pl.BlockSpec((C, N, N), lambda b: (b, 0, 0)),
        pl.BlockSpec((C, N, D), lambda b: (b, 0, 0)),
        pl.BlockSpec((C, 1, 128), lambda b: (b, 0, 0)),
    ] + [_full_spec(wi.shape) for wi in weights]

    out_specs = [
        pl.BlockSpec((C, N, D), lambda b: (b, 0, 0)),
        pl.BlockSpec((C, 1, 128), lambda b: (b, 0, 0)),
        pl.BlockSpec((C, 1, 128), lambda b: (b, 0, 0)),
    ]
    out_shape = [
        jax.ShapeDtypeStruct((B, N, D), jnp.float32),
        jax.ShapeDtypeStruct((B, 1, 128), jnp.float32),
        jax.ShapeDtypeStruct((B, 1, 128), jnp.float32),
    ]

    z, klp, epp = pl.pallas_call(
        _body,
        grid=(B // C,),
        in_specs=in_specs,
        out_specs=out_specs,
        out_shape=out_shape,
        compiler_params=pltpu.CompilerParams(
            dimension_semantics=("arbitrary",),
            vmem_limit_bytes=110 * 1024 * 1024,
        ),
    )(x, a, eps, vb, *weights)
    return (z, klp[:, 0, 0], epp[:, 0, 0])
